# baseline (speedup 1.0000x reference)
import functools

import jax
import jax.numpy as jnp
from jax import lax
from jax.experimental import pallas as pl
from jax.experimental.pallas import tpu as pltpu
from jax.experimental.pallas import tpu_sc as plsc

N_NODES = 10000
N_EDGES = 320000
D = 128
GAMMA = 10.0
C_SCALE = 6.0 / 127.0
DUMMY = N_NODES


EDGE_BLK = 20000


def _edge_feat_body(d_ref, dst_ref, w1_ref, b1_ref, w2_ref, b2_ref,
                    out_ref, idx_ref):
    d_row = d_ref[0]
    idx_ref[0] = jnp.where(
        d_row < 3.0, dst_ref[0], jnp.full_like(dst_ref[0], DUMMY)
    )

    d = jnp.transpose(d_row, (1, 0))
    centers = lax.broadcasted_iota(jnp.int32, (1, D), 1).astype(jnp.float32) * C_SCALE
    diff = d - centers
    rbf = jnp.exp((-GAMMA) * diff * diff)

    feats = []
    for k in range(3):
        h = jnp.maximum(
            jnp.dot(rbf, w1_ref[k], preferred_element_type=jnp.float32)
            + b1_ref[k, :][None, :],
            0.0,
        )
        f = (
            jnp.dot(h, w2_ref[k], preferred_element_type=jnp.float32)
            + b2_ref[k, :][None, :]
        )
        feats.append(f)

    m1 = d >= 3.0
    m2 = d >= 4.5
    out_ref[...] = jnp.where(m2, feats[2], jnp.where(m1, feats[1], feats[0]))


def _edge_feat(d_rows, dst_rows, w1, b1, w2, b2):
    grid = N_EDGES // EDGE_BLK
    return pl.pallas_call(
        _edge_feat_body,
        grid=(grid,),
        in_specs=[
            pl.BlockSpec((1, 1, EDGE_BLK), lambda i: (i, 0, 0)),
            pl.BlockSpec((1, 1, EDGE_BLK), lambda i: (i, 0, 0)),
            pl.BlockSpec((3, D, D), lambda i: (0, 0, 0)),
            pl.BlockSpec((3, D), lambda i: (0, 0)),
            pl.BlockSpec((3, D, D), lambda i: (0, 0, 0)),
            pl.BlockSpec((3, D), lambda i: (0, 0)),
        ],
        out_specs=[
            pl.BlockSpec((EDGE_BLK, D), lambda i: (i, 0)),
            pl.BlockSpec((1, 1, EDGE_BLK), lambda i: (i, 0, 0)),
        ],
        out_shape=[
            jax.ShapeDtypeStruct((N_EDGES, D), jnp.float32),
            jax.ShapeDtypeStruct((N_EDGES // EDGE_BLK, 1, EDGE_BLK), jnp.int32),
        ],
        compiler_params=pltpu.CompilerParams(
            dimension_semantics=("arbitrary",),
        ),
    )(d_rows, dst_rows, w1, b1, w2, b2)


NC, NS = 2, 16
NW = NC * NS
E_PER_W = N_EDGES // NW
CHUNK = 80
N_CHUNKS = E_PER_W // CHUNK
ACC_ROWS = 10008
OUT_ROWS = 624
NBUF = 4


def _seg_body(idx_hbm, feat_hbm, zeros_hbm, out_hbm,
              idx_v, feat_v, acc_s, gsem, isem, zsem):
    core = lax.axis_index("c")
    sid = lax.axis_index("s")
    wid = core * NS + sid
    base = wid * E_PER_W

    zbase = sid * OUT_ROWS
    pltpu.async_copy(
        zeros_hbm.at[pl.ds(zbase, OUT_ROWS)],
        acc_s.at[pl.ds(zbase, OUT_ROWS)],
        zsem,
    )

    @pl.when(sid == 0)
    def _ztail():
        pltpu.async_copy(
            zeros_hbm.at[pl.ds(NS * OUT_ROWS, N_NODES - NS * OUT_ROWS)],
            acc_s.at[pl.ds(NS * OUT_ROWS, N_NODES - NS * OUT_ROWS)],
            zsem,
        )

    pltpu.make_async_copy(
        zeros_hbm.at[pl.ds(zbase, OUT_ROWS)],
        acc_s.at[pl.ds(zbase, OUT_ROWS)],
        zsem,
    ).wait()

    @pl.when(sid == 0)
    def _ztailw():
        pltpu.make_async_copy(
            zeros_hbm.at[pl.ds(NS * OUT_ROWS, N_NODES - NS * OUT_ROWS)],
            acc_s.at[pl.ds(NS * OUT_ROWS, N_NODES - NS * OUT_ROWS)],
            zsem,
        ).wait()

    plsc.subcore_barrier()

    def _fetch(j, b):
        pltpu.async_copy(
            feat_hbm.at[pl.ds(base + j * CHUNK, CHUNK)],
            feat_v.at[b],
            gsem.at[b],
        )
        pltpu.async_copy(idx_hbm.at[wid, j], idx_v.at[b], isem.at[b])

    for b in range(NBUF):
        _fetch(b, b)

    def _visit(j, b):
        pltpu.make_async_copy(
            feat_hbm.at[pl.ds(base, CHUNK)], feat_v.at[b], gsem.at[b]
        ).wait()
        pltpu.make_async_copy(
            idx_hbm.at[wid, 0], idx_v.at[b], isem.at[b]
        ).wait()
        pltpu.sync_copy(feat_v.at[b], acc_s.at[idx_v.at[b]], add=True)

        @pl.when(j + NBUF < N_CHUNKS)
        def _next():
            _fetch(j + NBUF, b)

    @pl.loop(0, N_CHUNKS // NBUF)
    def _ring(g):
        for b in range(NBUF):
            _visit(g * NBUF + b, b)

    for j in range((N_CHUNKS // NBUF) * NBUF, N_CHUNKS):
        _visit(j, j % NBUF)

    plsc.subcore_barrier()

    pltpu.sync_copy(
        acc_s.at[pl.ds(zbase, OUT_ROWS)],
        out_hbm.at[core, pl.ds(zbase, OUT_ROWS)],
    )

    @pl.when(sid == 0)
    def _tail():
        pltpu.sync_copy(
            acc_s.at[pl.ds(NS * OUT_ROWS, N_NODES - NS * OUT_ROWS)],
            out_hbm.at[core, pl.ds(NS * OUT_ROWS, N_NODES - NS * OUT_ROWS)],
        )


@functools.partial(jax.jit, static_argnums=())
def _segment_sum_sc(idx_rows, edge_feat, zeros):
    mesh = plsc.VectorSubcoreMesh(core_axis_name="c", subcore_axis_name="s")
    f = pl.kernel(
        _seg_body,
        out_type=jax.ShapeDtypeStruct((NC, N_NODES, D), jnp.float32),
        mesh=mesh,
        scratch_types=[
            pltpu.VMEM((NBUF, CHUNK), jnp.int32),
            pltpu.VMEM((NBUF, CHUNK, D), jnp.float32),
            pltpu.VMEM_SHARED((ACC_ROWS, D), jnp.float32),
            pltpu.SemaphoreType.DMA((NBUF,)),
            pltpu.SemaphoreType.DMA((NBUF,)),
            pltpu.SemaphoreType.DMA,
        ],
    )
    return f(idx_rows, edge_feat, zeros)


def kernel(edge_lengths, edge_index, pos,
           W1_0, b1_0, W2_0, b2_0,
           W1_1, b1_1, W2_1, b2_1,
           W1_2, b1_2, W2_2, b2_2):
    w1 = jnp.stack([W1_0, W1_1, W1_2])
    b1 = jnp.stack([b1_0, b1_1, b1_2])
    w2 = jnp.stack([W2_0, W2_1, W2_2])
    b2 = jnp.stack([b2_0, b2_1, b2_2])
    d_rows = edge_lengths.reshape(N_EDGES // EDGE_BLK, 1, EDGE_BLK)
    dst_rows = edge_index[1].reshape(N_EDGES // EDGE_BLK, 1, EDGE_BLK)

    edge_feat, idx_rows = _edge_feat(d_rows, dst_rows, w1, b1, w2, b2)

    zeros = jnp.zeros((N_NODES, D), jnp.float32)
    partials = _segment_sum_sc(
        idx_rows.reshape(NW, N_CHUNKS, CHUNK), edge_feat, zeros
    )
    node_energy = partials[0] + partials[1]
    return edge_feat, node_energy


# speedup vs baseline: 1.0263x; 1.0263x over previous
import functools

import jax
import jax.numpy as jnp
from jax import lax
from jax.experimental import pallas as pl
from jax.experimental.pallas import tpu as pltpu
from jax.experimental.pallas import tpu_sc as plsc

N_NODES = 10000
N_EDGES = 320000
D = 128
GAMMA = 10.0
C_SCALE = 6.0 / 127.0
DUMMY = N_NODES


EDGE_BLK = 16000


def _edge_feat_body(d_ref, dst_ref, w1_ref, b1_ref, w2_ref, b2_ref,
                    out_ref, idx_ref):
    d_row = d_ref[0]
    idx_ref[0] = jnp.where(
        d_row < 3.0, dst_ref[0], jnp.full_like(dst_ref[0], DUMMY)
    )

    d = jnp.transpose(d_row, (1, 0))
    centers = lax.broadcasted_iota(jnp.int32, (1, D), 1).astype(jnp.float32) * C_SCALE
    diff = d - centers
    rbf = jnp.exp((-GAMMA) * diff * diff)

    feats = []
    for k in range(3):
        h = jnp.maximum(
            jnp.dot(rbf, w1_ref[k], preferred_element_type=jnp.float32)
            + b1_ref[k, :][None, :],
            0.0,
        )
        f = (
            jnp.dot(h, w2_ref[k], preferred_element_type=jnp.float32)
            + b2_ref[k, :][None, :]
        )
        feats.append(f)

    m1 = d >= 3.0
    m2 = d >= 4.5
    out_ref[...] = jnp.where(m2, feats[2], jnp.where(m1, feats[1], feats[0]))


def _edge_feat(d_rows, dst_rows, w1, b1, w2, b2):
    grid = N_EDGES // EDGE_BLK
    return pl.pallas_call(
        _edge_feat_body,
        grid=(grid,),
        in_specs=[
            pl.BlockSpec((1, 1, EDGE_BLK), lambda i: (i, 0, 0)),
            pl.BlockSpec((1, 1, EDGE_BLK), lambda i: (i, 0, 0)),
            pl.BlockSpec((3, D, D), lambda i: (0, 0, 0)),
            pl.BlockSpec((3, D), lambda i: (0, 0)),
            pl.BlockSpec((3, D, D), lambda i: (0, 0, 0)),
            pl.BlockSpec((3, D), lambda i: (0, 0)),
        ],
        out_specs=[
            pl.BlockSpec((EDGE_BLK, D), lambda i: (i, 0)),
            pl.BlockSpec((1, 1, EDGE_BLK), lambda i: (i, 0, 0)),
        ],
        out_shape=[
            jax.ShapeDtypeStruct((N_EDGES, D), jnp.float32),
            jax.ShapeDtypeStruct((N_EDGES // EDGE_BLK, 1, EDGE_BLK), jnp.int32),
        ],
        compiler_params=pltpu.CompilerParams(
            dimension_semantics=("arbitrary",),
        ),
    )(d_rows, dst_rows, w1, b1, w2, b2)


NC, NS = 2, 16
NW = NC * NS
E_PER_W = N_EDGES // NW
CHUNK = 128
N_CHUNKS = 79
TAIL_OFF = E_PER_W - CHUNK
ACC_ROWS = 10008
OUT_ROWS = 624
NBUF = 3


def _seg_body(idx_hbm, feat_hbm, zeros_hbm, out_hbm,
              idx_v, feat_v, acc_s, gsem, isem, zsem):
    core = lax.axis_index("c")
    sid = lax.axis_index("s")
    wid = core * NS + sid
    base = wid * E_PER_W

    zbase = sid * OUT_ROWS
    pltpu.async_copy(
        zeros_hbm.at[pl.ds(zbase, OUT_ROWS)],
        acc_s.at[pl.ds(zbase, OUT_ROWS)],
        zsem,
    )

    @pl.when(sid == 0)
    def _ztail():
        pltpu.async_copy(
            zeros_hbm.at[pl.ds(NS * OUT_ROWS, N_NODES - NS * OUT_ROWS)],
            acc_s.at[pl.ds(NS * OUT_ROWS, N_NODES - NS * OUT_ROWS)],
            zsem,
        )

    pltpu.make_async_copy(
        zeros_hbm.at[pl.ds(zbase, OUT_ROWS)],
        acc_s.at[pl.ds(zbase, OUT_ROWS)],
        zsem,
    ).wait()

    @pl.when(sid == 0)
    def _ztailw():
        pltpu.make_async_copy(
            zeros_hbm.at[pl.ds(NS * OUT_ROWS, N_NODES - NS * OUT_ROWS)],
            acc_s.at[pl.ds(NS * OUT_ROWS, N_NODES - NS * OUT_ROWS)],
            zsem,
        ).wait()

    plsc.subcore_barrier()

    def _fetch(j, b):
        start = base + jnp.minimum(j * CHUNK, TAIL_OFF)
        pltpu.async_copy(
            feat_hbm.at[pl.ds(start, CHUNK)],
            feat_v.at[b],
            gsem.at[b],
        )
        pltpu.async_copy(idx_hbm.at[wid, j], idx_v.at[b], isem.at[b])

    for b in range(NBUF):
        _fetch(b, b)

    def _visit(j, b):
        pltpu.make_async_copy(
            feat_hbm.at[pl.ds(base, CHUNK)], feat_v.at[b], gsem.at[b]
        ).wait()
        pltpu.make_async_copy(
            idx_hbm.at[wid, 0], idx_v.at[b], isem.at[b]
        ).wait()
        pltpu.sync_copy(feat_v.at[b], acc_s.at[idx_v.at[b]], add=True)

        @pl.when(j + NBUF < N_CHUNKS)
        def _next():
            _fetch(j + NBUF, b)

    @pl.loop(0, N_CHUNKS // NBUF)
    def _ring(g):
        for b in range(NBUF):
            _visit(g * NBUF + b, b)

    for j in range((N_CHUNKS // NBUF) * NBUF, N_CHUNKS):
        _visit(j, j % NBUF)

    plsc.subcore_barrier()

    pltpu.sync_copy(
        acc_s.at[pl.ds(zbase, OUT_ROWS)],
        out_hbm.at[core, pl.ds(zbase, OUT_ROWS)],
    )

    @pl.when(sid == 0)
    def _tail():
        pltpu.sync_copy(
            acc_s.at[pl.ds(NS * OUT_ROWS, N_NODES - NS * OUT_ROWS)],
            out_hbm.at[core, pl.ds(NS * OUT_ROWS, N_NODES - NS * OUT_ROWS)],
        )


@functools.partial(jax.jit, static_argnums=())
def _segment_sum_sc(idx_rows, edge_feat, zeros):
    mesh = plsc.VectorSubcoreMesh(core_axis_name="c", subcore_axis_name="s")
    f = pl.kernel(
        _seg_body,
        out_type=jax.ShapeDtypeStruct((NC, N_NODES, D), jnp.float32),
        mesh=mesh,
        scratch_types=[
            pltpu.VMEM((NBUF, CHUNK), jnp.int32),
            pltpu.VMEM((NBUF, CHUNK, D), jnp.float32),
            pltpu.VMEM_SHARED((ACC_ROWS, D), jnp.float32),
            pltpu.SemaphoreType.DMA((NBUF,)),
            pltpu.SemaphoreType.DMA((NBUF,)),
            pltpu.SemaphoreType.DMA,
        ],
    )
    return f(idx_rows, edge_feat, zeros)


def kernel(edge_lengths, edge_index, pos,
           W1_0, b1_0, W2_0, b2_0,
           W1_1, b1_1, W2_1, b2_1,
           W1_2, b1_2, W2_2, b2_2):
    w1 = jnp.stack([W1_0, W1_1, W1_2])
    b1 = jnp.stack([b1_0, b1_1, b1_2])
    w2 = jnp.stack([W2_0, W2_1, W2_2])
    b2 = jnp.stack([b2_0, b2_1, b2_2])
    d_rows = edge_lengths.reshape(N_EDGES // EDGE_BLK, 1, EDGE_BLK)
    dst_rows = edge_index[1].reshape(N_EDGES // EDGE_BLK, 1, EDGE_BLK)

    edge_feat, idx_rows = _edge_feat(d_rows, dst_rows, w1, b1, w2, b2)

    zeros = jnp.zeros((N_NODES, D), jnp.float32)
    idx2 = idx_rows.reshape(NW, E_PER_W)
    main = idx2[:, : 78 * CHUNK].reshape(NW, 78, CHUNK)
    tail = jnp.concatenate(
        [
            jnp.full((NW, 78 * CHUNK - TAIL_OFF), DUMMY, jnp.int32),
            idx2[:, 78 * CHUNK:],
        ],
        axis=1,
    ).reshape(NW, 1, CHUNK)
    idx_chunks = jnp.concatenate([main, tail], axis=1)
    partials = _segment_sum_sc(idx_chunks, edge_feat, zeros)
    node_energy = partials[0] + partials[1]
    return edge_feat, node_energy


# speedup vs baseline: 1.0300x; 1.0036x over previous
import functools

import jax
import jax.numpy as jnp
from jax import lax
from jax.experimental import pallas as pl
from jax.experimental.pallas import tpu as pltpu
from jax.experimental.pallas import tpu_sc as plsc

N_NODES = 10000
N_EDGES = 320000
D = 128
GAMMA = 10.0
C_SCALE = 6.0 / 127.0
DUMMY = N_NODES


EDGE_BLK = 16000


def _edge_feat_body(d_ref, dst_ref, w1_ref, b1_ref, w2_ref, b2_ref,
                    out_ref, idx_ref):
    d_row = d_ref[0]
    idx_ref[0] = jnp.where(
        d_row < 3.0, dst_ref[0], jnp.full_like(dst_ref[0], DUMMY)
    )

    d = jnp.transpose(d_row, (1, 0))
    centers = lax.broadcasted_iota(jnp.int32, (1, D), 1).astype(jnp.float32) * C_SCALE
    diff = d - centers
    rbf = jnp.exp((-GAMMA) * diff * diff)

    feats = []
    for k in range(3):
        h = jnp.maximum(
            jnp.dot(rbf, w1_ref[k], preferred_element_type=jnp.float32)
            + b1_ref[k, :][None, :],
            0.0,
        )
        f = (
            jnp.dot(h, w2_ref[k], preferred_element_type=jnp.float32)
            + b2_ref[k, :][None, :]
        )
        feats.append(f)

    m1 = d >= 3.0
    m2 = d >= 4.5
    out_ref[...] = jnp.where(m2, feats[2], jnp.where(m1, feats[1], feats[0]))


def _edge_feat(d_rows, dst_rows, w1, b1, w2, b2):
    grid = N_EDGES // EDGE_BLK
    return pl.pallas_call(
        _edge_feat_body,
        grid=(grid,),
        in_specs=[
            pl.BlockSpec((1, 1, EDGE_BLK), lambda i: (i, 0, 0)),
            pl.BlockSpec((1, 1, EDGE_BLK), lambda i: (i, 0, 0)),
            pl.BlockSpec((3, D, D), lambda i: (0, 0, 0)),
            pl.BlockSpec((3, D), lambda i: (0, 0)),
            pl.BlockSpec((3, D, D), lambda i: (0, 0, 0)),
            pl.BlockSpec((3, D), lambda i: (0, 0)),
        ],
        out_specs=[
            pl.BlockSpec((EDGE_BLK, D), lambda i: (i, 0)),
            pl.BlockSpec((1, 1, EDGE_BLK), lambda i: (i, 0, 0)),
        ],
        out_shape=[
            jax.ShapeDtypeStruct((N_EDGES, D), jnp.float32),
            jax.ShapeDtypeStruct((N_EDGES // EDGE_BLK, 1, EDGE_BLK), jnp.int32),
        ],
        compiler_params=pltpu.CompilerParams(
            dimension_semantics=("arbitrary",),
        ),
    )(d_rows, dst_rows, w1, b1, w2, b2)


NC, NS = 2, 16
NW = NC * NS
E_PER_W = N_EDGES // NW
CHUNK = 80
N_CHUNKS = E_PER_W // CHUNK
ACC_ROWS = 10008
OUT_ROWS = 624
NBUF = 4


def _seg_body(idx_hbm, feat_hbm, zeros_hbm, out_hbm,
              idx_v, feat_v, acc_s, gsem, isem, zsem):
    core = lax.axis_index("c")
    sid = lax.axis_index("s")
    wid = core * NS + sid
    base = wid * E_PER_W

    zbase = sid * OUT_ROWS
    pltpu.async_copy(
        zeros_hbm.at[pl.ds(zbase, OUT_ROWS)],
        acc_s.at[pl.ds(zbase, OUT_ROWS)],
        zsem,
    )

    @pl.when(sid == 0)
    def _ztail():
        pltpu.async_copy(
            zeros_hbm.at[pl.ds(NS * OUT_ROWS, N_NODES - NS * OUT_ROWS)],
            acc_s.at[pl.ds(NS * OUT_ROWS, N_NODES - NS * OUT_ROWS)],
            zsem,
        )

    pltpu.make_async_copy(
        zeros_hbm.at[pl.ds(zbase, OUT_ROWS)],
        acc_s.at[pl.ds(zbase, OUT_ROWS)],
        zsem,
    ).wait()

    @pl.when(sid == 0)
    def _ztailw():
        pltpu.make_async_copy(
            zeros_hbm.at[pl.ds(NS * OUT_ROWS, N_NODES - NS * OUT_ROWS)],
            acc_s.at[pl.ds(NS * OUT_ROWS, N_NODES - NS * OUT_ROWS)],
            zsem,
        ).wait()

    plsc.subcore_barrier()

    def _fetch(j, b):
        pltpu.async_copy(
            feat_hbm.at[pl.ds(base + j * CHUNK, CHUNK)],
            feat_v.at[b],
            gsem.at[b],
        )
        pltpu.async_copy(idx_hbm.at[wid, j], idx_v.at[b], isem.at[b])

    for b in range(NBUF):
        _fetch(b, b)

    def _visit(j, b):
        pltpu.make_async_copy(
            feat_hbm.at[pl.ds(base, CHUNK)], feat_v.at[b], gsem.at[b]
        ).wait()
        pltpu.make_async_copy(
            idx_hbm.at[wid, 0], idx_v.at[b], isem.at[b]
        ).wait()
        pltpu.sync_copy(feat_v.at[b], acc_s.at[idx_v.at[b]], add=True)

        @pl.when(j + NBUF < N_CHUNKS)
        def _next():
            _fetch(j + NBUF, b)

    @pl.loop(0, N_CHUNKS // NBUF)
    def _ring(g):
        for b in range(NBUF):
            _visit(g * NBUF + b, b)

    for j in range((N_CHUNKS // NBUF) * NBUF, N_CHUNKS):
        _visit(j, j % NBUF)

    plsc.subcore_barrier()

    pltpu.sync_copy(
        acc_s.at[pl.ds(zbase, OUT_ROWS)],
        out_hbm.at[core, pl.ds(zbase, OUT_ROWS)],
    )

    @pl.when(sid == 0)
    def _tail():
        pltpu.sync_copy(
            acc_s.at[pl.ds(NS * OUT_ROWS, N_NODES - NS * OUT_ROWS)],
            out_hbm.at[core, pl.ds(NS * OUT_ROWS, N_NODES - NS * OUT_ROWS)],
        )


@functools.partial(jax.jit, static_argnums=())
def _segment_sum_sc(idx_rows, edge_feat, zeros):
    mesh = plsc.VectorSubcoreMesh(core_axis_name="c", subcore_axis_name="s")
    f = pl.kernel(
        _seg_body,
        out_type=jax.ShapeDtypeStruct((NC, N_NODES, D), jnp.float32),
        mesh=mesh,
        scratch_types=[
            pltpu.VMEM((NBUF, CHUNK), jnp.int32),
            pltpu.VMEM((NBUF, CHUNK, D), jnp.float32),
            pltpu.VMEM_SHARED((ACC_ROWS, D), jnp.float32),
            pltpu.SemaphoreType.DMA((NBUF,)),
            pltpu.SemaphoreType.DMA((NBUF,)),
            pltpu.SemaphoreType.DMA,
        ],
    )
    return f(idx_rows, edge_feat, zeros)


def kernel(edge_lengths, edge_index, pos,
           W1_0, b1_0, W2_0, b2_0,
           W1_1, b1_1, W2_1, b2_1,
           W1_2, b1_2, W2_2, b2_2):
    w1 = jnp.stack([W1_0, W1_1, W1_2])
    b1 = jnp.stack([b1_0, b1_1, b1_2])
    w2 = jnp.stack([W2_0, W2_1, W2_2])
    b2 = jnp.stack([b2_0, b2_1, b2_2])
    d_rows = edge_lengths.reshape(N_EDGES // EDGE_BLK, 1, EDGE_BLK)
    dst_rows = edge_index[1].reshape(N_EDGES // EDGE_BLK, 1, EDGE_BLK)

    edge_feat, idx_rows = _edge_feat(d_rows, dst_rows, w1, b1, w2, b2)

    zeros = jnp.zeros((N_NODES, D), jnp.float32)
    partials = _segment_sum_sc(
        idx_rows.reshape(NW, N_CHUNKS, CHUNK), edge_feat, zeros
    )
    node_energy = partials[0] + partials[1]
    return edge_feat, node_energy
